# Initial kernel scaffold; baseline (speedup 1.0000x reference)
import jax
import jax.numpy as jnp
from jax.experimental import pallas as pl


def kernel(x, y, mask):
    raise NotImplementedError("write your pallas kernel here")


# speedup vs baseline: 5.4150x; 5.4150x over previous
import functools
import math

import jax
import jax.numpy as jnp
from jax.experimental import pallas as pl
from jax.experimental.pallas import tpu as pltpu

B = 4096
C = 4096
K = math.ceil(B * C * 0.2)
BLK_R = 128
NB = B // BLK_R
NPASS = 8
NPIV = 15
INF_BITS = 0x7F800000


def _bce(x, y):
    return jnp.maximum(x, 0.0) - x * y + jnp.log1p(jnp.exp(-jnp.abs(x)))


def _bits_kernel(x_ref, y_ref, m_ref, bits_ref):
    loss = _bce(x_ref[...], y_ref[...])
    b = pltpu.bitcast(loss, jnp.int32)
    bits_ref[...] = jnp.where(m_ref[...] == 0, b, jnp.int32(-1))


def _search_kernel(bits_ref, thr_ref, state):
    s = pl.program_id(0)
    b = pl.program_id(1)

    @pl.when(jnp.logical_and(s == 0, b == 0))
    def _init():
        state[0] = jnp.int32(0)
        state[1] = jnp.int32(INF_BITS)
        for j in range(NPIV):
            state[2 + j] = jnp.int32(0)

    @pl.when(jnp.logical_and(s > 0, b == 0))
    def _update():
        lo = state[0]
        hi = state[1]
        step = jnp.maximum((hi - lo) // (NPIV + 1), 1)
        new_lo = lo
        new_hi = hi
        for j in range(NPIV):
            pj = lo + step * (j + 1)
            cj = state[2 + j]
            ok = jnp.logical_and(pj < hi, cj >= K)
            new_lo = jnp.where(ok, jnp.maximum(new_lo, pj), new_lo)
            bad = jnp.logical_and(pj < hi, cj < K)
            new_hi = jnp.where(bad, jnp.minimum(new_hi, pj), new_hi)
            state[2 + j] = jnp.int32(0)
        state[0] = new_lo
        state[1] = new_hi

    lo = state[0]
    hi = state[1]
    step = jnp.maximum((hi - lo) // (NPIV + 1), 1)
    bits = bits_ref[...]
    for j in range(NPIV):
        pj = lo + step * (j + 1)
        state[2 + j] += jnp.sum((bits >= pj).astype(jnp.int32))

    @pl.when(jnp.logical_and(s == NPASS - 1, b == NB - 1))
    def _finish():
        flo = state[0]
        fhi = state[1]
        fstep = jnp.maximum((fhi - flo) // (NPIV + 1), 1)
        res = flo
        for j in range(NPIV):
            pj = flo + fstep * (j + 1)
            ok = jnp.logical_and(pj < fhi, state[2 + j] >= K)
            res = jnp.where(ok, jnp.maximum(res, pj), res)
        thr_ref[0, 0] = res


def _apply_kernel(x_ref, y_ref, m_ref, bits_ref, thr_ref, out_ref, cnt_ref,
                  off_ref, carry):
    b = pl.program_id(0)

    @pl.when(b == 0)
    def _init():
        carry[0] = jnp.int32(0)

    x = x_ref[...]
    y = y_ref[...]
    orig = _bce(x, y)
    corr = _bce(x, 1.0 - y)
    cond = bits_ref[...] >= thr_ref[0, 0]
    out_ref[...] = jnp.where(cond, corr, orig)

    cnt = jnp.sum(cond.astype(jnp.int32), axis=1)
    r = jax.lax.broadcasted_iota(jnp.int32, (BLK_R, BLK_R), 0)
    c = jax.lax.broadcasted_iota(jnp.int32, (BLK_R, BLK_R), 1)
    tril = (c < r).astype(jnp.float32)
    excl = jax.lax.dot(tril, cnt.astype(jnp.float32).reshape(BLK_R, 1),
                       preferred_element_type=jnp.float32)
    off = excl.reshape(BLK_R).astype(jnp.int32) + carry[0]
    cnt_ref[...] = cnt.reshape(1, 1, BLK_R)
    off_ref[...] = off.reshape(1, 1, BLK_R)
    carry[0] += jnp.sum(cnt)


def _build_dense(x, y, mask):
    bits = pl.pallas_call(
        _bits_kernel,
        grid=(NB,),
        in_specs=[
            pl.BlockSpec((BLK_R, C), lambda b: (b, 0)),
            pl.BlockSpec((BLK_R, C), lambda b: (b, 0)),
            pl.BlockSpec((BLK_R, C), lambda b: (b, 0)),
        ],
        out_specs=pl.BlockSpec((BLK_R, C), lambda b: (b, 0)),
        out_shape=jax.ShapeDtypeStruct((B, C), jnp.int32),
    )(x, y, mask)

    thr = pl.pallas_call(
        _search_kernel,
        grid=(NPASS, NB),
        in_specs=[pl.BlockSpec((BLK_R, C), lambda s, b: (b, 0))],
        out_specs=pl.BlockSpec(memory_space=pltpu.SMEM),
        out_shape=jax.ShapeDtypeStruct((1, 1), jnp.int32),
        scratch_shapes=[pltpu.SMEM((2 + NPIV,), jnp.int32)],
    )(bits)

    out_loss, cnt3, off3 = pl.pallas_call(
        _apply_kernel,
        grid=(NB,),
        in_specs=[
            pl.BlockSpec((BLK_R, C), lambda b: (b, 0)),
            pl.BlockSpec((BLK_R, C), lambda b: (b, 0)),
            pl.BlockSpec((BLK_R, C), lambda b: (b, 0)),
            pl.BlockSpec((BLK_R, C), lambda b: (b, 0)),
            pl.BlockSpec(memory_space=pltpu.SMEM),
        ],
        out_specs=[
            pl.BlockSpec((BLK_R, C), lambda b: (b, 0)),
            pl.BlockSpec((1, 1, BLK_R), lambda b: (b, 0, 0)),
            pl.BlockSpec((1, 1, BLK_R), lambda b: (b, 0, 0)),
        ],
        out_shape=[
            jax.ShapeDtypeStruct((B, C), jnp.float32),
            jax.ShapeDtypeStruct((NB, 1, BLK_R), jnp.int32),
            jax.ShapeDtypeStruct((NB, 1, BLK_R), jnp.int32),
        ],
        scratch_shapes=[pltpu.SMEM((1,), jnp.int32)],
    )(x, y, mask, bits, thr)

    return bits, thr, out_loss, cnt3, off3


@jax.jit
def kernel(x, y, mask):
    bits, thr, out_loss, cnt3, off3 = _build_dense(x, y, mask)
    cond = bits >= thr[0, 0]
    idx0, idx1 = jnp.nonzero(cond, size=K, fill_value=0)
    return out_loss, idx0.astype(jnp.int32), idx1.astype(jnp.int32)
